# Initial kernel scaffold; baseline (speedup 1.0000x reference)
import jax
import jax.numpy as jnp
from jax.experimental import pallas as pl


def kernel(idx, table, W, b):
    raise NotImplementedError("write your pallas kernel here")


# speedup vs baseline: 10.8308x; 10.8308x over previous
import functools

import jax
import jax.numpy as jnp
from jax import lax
from jax.experimental import pallas as pl
from jax.experimental.pallas import tpu as pltpu
from jax.experimental.pallas import tpu_sc as plsc


def _sc_gather(table, idx_flat, chunk=1024):
    B = idx_flat.shape[0]
    V, D = table.shape
    info = plsc.get_sparse_core_info()
    nw = info.num_cores * info.num_subcores
    b_per_w = B // nw
    n_chunks = b_per_w // chunk
    mesh = plsc.VectorSubcoreMesh(core_axis_name="c", subcore_axis_name="s")

    @functools.partial(
        pl.kernel,
        mesh=mesh,
        out_type=jax.ShapeDtypeStruct((B, D), jnp.float32),
        scratch_types=[
            pltpu.VMEM((chunk,), jnp.int32),
            pltpu.VMEM((chunk, D), jnp.float32),
            pltpu.SemaphoreType.DMA,
        ],
        compiler_params=pltpu.CompilerParams(use_tc_tiling_on_sc=False),
    )
    def gather_kernel(table_hbm, idx_hbm, out_hbm, idx_v, rows_v, sem):
        wid = lax.axis_index("s") * info.num_cores + lax.axis_index("c")
        base = wid * b_per_w

        def body(i, carry):
            off = base + i * chunk
            pltpu.sync_copy(idx_hbm.at[pl.ds(off, chunk)], idx_v)
            pltpu.async_copy(table_hbm.at[idx_v], rows_v, sem).wait()
            pltpu.sync_copy(rows_v, out_hbm.at[pl.ds(off, chunk)])
            return carry

        lax.fori_loop(0, n_chunks, body, 0)

    return gather_kernel(table, idx_flat)


def _tc_project(emb, w_t, bias, block_m=4096):
    B, D = emb.shape
    O = w_t.shape[1]

    def body(emb_ref, wt_ref, b_ref, out_ref):
        out_ref[...] = (
            jnp.dot(emb_ref[...], wt_ref[...], preferred_element_type=jnp.float32)
            + b_ref[...]
        )

    return pl.pallas_call(
        body,
        grid=(B // block_m,),
        in_specs=[
            pl.BlockSpec((block_m, D), lambda i: (i, 0)),
            pl.BlockSpec((D, O), lambda i: (0, 0)),
            pl.BlockSpec((1, O), lambda i: (0, 0)),
        ],
        out_specs=pl.BlockSpec((block_m, O), lambda i: (i, 0)),
        out_shape=jax.ShapeDtypeStruct((B, O), jnp.float32),
    )(emb, w_t, bias.reshape(1, O))


def kernel(idx, table, W, b):
    Bb, L = idx.shape
    O = W.shape[0]
    idx_flat = idx.reshape(-1).astype(jnp.int32)
    emb = _sc_gather(table, idx_flat)
    out = _tc_project(emb, W.T, b)
    return out.reshape(Bb, L, O)


# speedup vs baseline: 13.4688x; 1.2436x over previous
import functools

import jax
import jax.numpy as jnp
from jax import lax
from jax.experimental import pallas as pl
from jax.experimental.pallas import tpu as pltpu
from jax.experimental.pallas import tpu_sc as plsc


def _sc_gather(table, idx_flat, chunk=640):
    B = idx_flat.shape[0]
    V, D = table.shape
    Q = B // 4
    info = plsc.get_sparse_core_info()
    nw = info.num_cores * info.num_subcores
    g_per_w = Q // nw
    n_chunks = g_per_w // chunk
    mesh = plsc.VectorSubcoreMesh(core_axis_name="c", subcore_axis_name="s")

    @functools.partial(
        pl.kernel,
        mesh=mesh,
        out_type=jax.ShapeDtypeStruct((Q, 4 * D), jnp.float32),
        scratch_types=[
            pltpu.VMEM((4 * chunk,), jnp.int32),
            pltpu.VMEM((4 * chunk, D), jnp.float32),
            pltpu.SemaphoreType.DMA,
        ],
        compiler_params=pltpu.CompilerParams(use_tc_tiling_on_sc=False),
    )
    def gather_kernel(table_hbm, idx_hbm, out_hbm, idx_v, rows_v, sem):
        wid = lax.axis_index("s") * info.num_cores + lax.axis_index("c")
        base = wid * g_per_w

        def body(i, carry):
            g0 = base + i * chunk
            for r in range(4):
                pltpu.sync_copy(
                    idx_hbm.at[pl.ds(r * Q + g0, chunk)],
                    idx_v.at[pl.ds(r * chunk, chunk)],
                )
            copies = [
                pltpu.async_copy(
                    table_hbm.at[idx_v.at[pl.ds(r * chunk, chunk)]],
                    rows_v.at[pl.ds(r * chunk, chunk)],
                    sem,
                )
                for r in range(4)
            ]
            for c in copies:
                c.wait()
            for r in range(4):
                pltpu.sync_copy(
                    rows_v.at[pl.ds(r * chunk, chunk)],
                    out_hbm.at[pl.ds(g0, chunk), pl.ds(r * D, D)],
                )
            return carry

        lax.fori_loop(0, n_chunks, body, 0)

    return gather_kernel(table, idx_flat)


def _tc_project(emb2, w_t, bias, block_m=2048):
    Q = emb2.shape[0]
    D, O = w_t.shape

    def body(emb_ref, wt_ref, b_ref, out_ref):
        for r in range(4):
            out_ref[r] = (
                jnp.dot(
                    emb_ref[:, r * D : (r + 1) * D],
                    wt_ref[...],
                    preferred_element_type=jnp.float32,
                )
                + b_ref[...]
            )

    return pl.pallas_call(
        body,
        grid=(Q // block_m,),
        in_specs=[
            pl.BlockSpec((block_m, 4 * D), lambda i: (i, 0)),
            pl.BlockSpec((D, O), lambda i: (0, 0)),
            pl.BlockSpec((1, O), lambda i: (0, 0)),
        ],
        out_specs=pl.BlockSpec((4, block_m, O), lambda i: (0, i, 0)),
        out_shape=jax.ShapeDtypeStruct((4, Q, O), jnp.float32),
    )(emb2, w_t, bias.reshape(1, O))


def kernel(idx, table, W, b):
    Bb, L = idx.shape
    O = W.shape[0]
    idx_flat = idx.reshape(-1).astype(jnp.int32)
    emb2 = _sc_gather(table, idx_flat)
    out = _tc_project(emb2, W.T, b)
    return out.reshape(Bb * L, O).reshape(Bb, L, O)


# speedup vs baseline: 26.5632x; 1.9722x over previous
import functools

import jax
import jax.numpy as jnp
from jax import lax
from jax.experimental import pallas as pl
from jax.experimental.pallas import tpu as pltpu
from jax.experimental.pallas import tpu_sc as plsc


def _sc_gather(table, idx_flat, chunk=640):
    B = idx_flat.shape[0]
    V, D = table.shape
    Q = B // 4
    info = plsc.get_sparse_core_info()
    nw = info.num_cores * info.num_subcores
    g_per_w = Q // nw
    n_chunks = g_per_w // chunk
    mesh = plsc.VectorSubcoreMesh(core_axis_name="c", subcore_axis_name="s")

    @functools.partial(
        pl.kernel,
        mesh=mesh,
        out_type=jax.ShapeDtypeStruct((Q, 4 * D), jnp.float32),
        scratch_types=[
            pltpu.VMEM((4 * chunk,), jnp.int32),
            pltpu.VMEM((4 * chunk, D), jnp.float32),
            pltpu.SemaphoreType.DMA,
        ],
        compiler_params=pltpu.CompilerParams(use_tc_tiling_on_sc=False),
    )
    def gather_kernel(table_hbm, idx_hbm, out_hbm, idx_v, rows_v, sem):
        wid = lax.axis_index("s") * info.num_cores + lax.axis_index("c")
        base = wid * g_per_w

        def body(i, carry):
            g0 = base + i * chunk
            for r in range(4):
                pltpu.sync_copy(
                    idx_hbm.at[pl.ds(r * Q + g0, chunk)],
                    idx_v.at[pl.ds(r * chunk, chunk)],
                )
            copies = [
                pltpu.async_copy(
                    table_hbm.at[idx_v.at[pl.ds(r * chunk, chunk)]],
                    rows_v.at[pl.ds(r * chunk, chunk)],
                    sem,
                )
                for r in range(4)
            ]
            for c in copies:
                c.wait()
            for r in range(4):
                pltpu.sync_copy(
                    rows_v.at[pl.ds(r * chunk, chunk)],
                    out_hbm.at[pl.ds(g0, chunk), pl.ds(r * D, D)],
                )
            return carry

        lax.fori_loop(0, n_chunks, body, 0)

    return gather_kernel(table, idx_flat)


def _tc_project(emb2, w_t, bias, block_m=2048):
    Q = emb2.shape[0]
    D, O = w_t.shape

    def body(emb_ref, wt_ref, b_ref, out_ref):
        for r in range(4):
            out_ref[r] = (
                jnp.dot(
                    emb_ref[:, r * D : (r + 1) * D],
                    wt_ref[...],
                    preferred_element_type=jnp.float32,
                )
                + b_ref[...]
            )

    return pl.pallas_call(
        body,
        grid=(Q // block_m,),
        in_specs=[
            pl.BlockSpec((block_m, 4 * D), lambda i: (i, 0)),
            pl.BlockSpec((D, O), lambda i: (0, 0)),
            pl.BlockSpec((1, O), lambda i: (0, 0)),
        ],
        out_specs=pl.BlockSpec((4, block_m, O), lambda i: (0, i, 0)),
        out_shape=jax.ShapeDtypeStruct((4, Q, O), jnp.float32),
    )(emb2, w_t, bias.reshape(1, O))


def kernel(idx, table, W, b):
    Bb, L = idx.shape
    O = W.shape[0]
    idx_perm = idx.T.reshape(-1).astype(jnp.int32)
    emb2 = _sc_gather(table, idx_perm)
    out = _tc_project(emb2, W.T, b)
    return out.reshape(L, Bb, O).transpose(1, 0, 2)


# speedup vs baseline: 27.5012x; 1.0353x over previous
import functools

import jax
import jax.numpy as jnp
from jax import lax
from jax.experimental import pallas as pl
from jax.experimental.pallas import tpu as pltpu
from jax.experimental.pallas import tpu_sc as plsc


def _sc_gather(table, idx_flat, chunk=640):
    B = idx_flat.shape[0]
    V, D = table.shape
    Q = B // 4
    info = plsc.get_sparse_core_info()
    nw = info.num_cores * info.num_subcores
    g_per_w = Q // nw
    n_chunks = g_per_w // chunk
    mesh = plsc.VectorSubcoreMesh(core_axis_name="c", subcore_axis_name="s")

    @functools.partial(
        pl.kernel,
        mesh=mesh,
        out_type=jax.ShapeDtypeStruct((Q, 4 * D), jnp.float32),
        scratch_types=[
            pltpu.VMEM((4 * chunk,), jnp.int32),
            pltpu.VMEM((4 * chunk, D), jnp.float32),
            pltpu.SemaphoreType.DMA,
        ],
        compiler_params=pltpu.CompilerParams(use_tc_tiling_on_sc=False),
    )
    def gather_kernel(table_hbm, idx_hbm, out_hbm, idx_v, rows_v, sem):
        wid = lax.axis_index("s") * info.num_cores + lax.axis_index("c")
        base = wid * g_per_w

        def body(i, carry):
            g0 = base + i * chunk
            for r in range(4):
                pltpu.sync_copy(
                    idx_hbm.at[pl.ds(r * Q + g0, chunk)],
                    idx_v.at[pl.ds(r * chunk, chunk)],
                )
            copies = [
                pltpu.async_copy(
                    table_hbm.at[idx_v.at[pl.ds(r * chunk, chunk)]],
                    rows_v.at[pl.ds(r * chunk, chunk)],
                    sem,
                )
                for r in range(4)
            ]
            for c in copies:
                c.wait()
            for r in range(4):
                pltpu.sync_copy(
                    rows_v.at[pl.ds(r * chunk, chunk)],
                    out_hbm.at[pl.ds(g0, chunk), pl.ds(r * D, D)],
                )
            return carry

        lax.fori_loop(0, n_chunks, body, 0)

    return gather_kernel(table, idx_flat)


def _tc_project(emb2, w_t, bias, block_m=4096):
    Q = emb2.shape[0]
    D, O = w_t.shape

    def body(emb_ref, wt_ref, b_ref, out_ref):
        for r in range(4):
            out_ref[r] = (
                jnp.dot(
                    emb_ref[:, r * D : (r + 1) * D],
                    wt_ref[...],
                    preferred_element_type=jnp.float32,
                )
                + b_ref[...]
            )

    return pl.pallas_call(
        body,
        grid=(Q // block_m,),
        in_specs=[
            pl.BlockSpec((block_m, 4 * D), lambda i: (i, 0)),
            pl.BlockSpec((D, O), lambda i: (0, 0)),
            pl.BlockSpec((1, O), lambda i: (0, 0)),
        ],
        out_specs=pl.BlockSpec((4, block_m, O), lambda i: (0, i, 0)),
        out_shape=jax.ShapeDtypeStruct((4, Q, O), jnp.float32),
    )(emb2, w_t, bias.reshape(1, O))


def kernel(idx, table, W, b):
    Bb, L = idx.shape
    O = W.shape[0]
    idx_perm = idx.T.reshape(-1).astype(jnp.int32)
    emb2 = _sc_gather(table, idx_perm)
    out = _tc_project(emb2, W.T, b)
    return out.reshape(L, Bb, O).transpose(1, 0, 2)
